# Initial kernel scaffold; baseline (speedup 1.0000x reference)
import jax
import jax.numpy as jnp
from jax.experimental import pallas as pl


def kernel(atomic_embedding, pairlist, f_ij_cutoff, r_ij, W, b):
    raise NotImplementedError("write your pallas kernel here")


# speedup vs baseline: 20.4957x; 20.4957x over previous
import functools

import jax
import jax.numpy as jnp
from jax import lax
from jax.experimental import pallas as pl
from jax.experimental.pallas import tpu as pltpu
from jax.experimental.pallas import tpu_sc as plsc

N = 10000
E = 160000
D = 128

NC = 2
NS = 16
NW = NC * NS


_EPT = E // NW
_GC = 128
_GN = _EPT // _GC
_GT = _EPT - _GN * _GC

_sc_mesh = plsc.VectorSubcoreMesh(core_axis_name="c", subcore_axis_name="s")


@functools.partial(
    pl.kernel,
    mesh=_sc_mesh,
    out_type=jax.ShapeDtypeStruct((E, D), jnp.float32),
    scratch_types=[
        pltpu.VMEM((_GC,), jnp.int32),
        pltpu.VMEM((_GT,), jnp.int32),
        pltpu.VMEM((_GC, D), jnp.float32),
        pltpu.SemaphoreType.DMA,
    ],
)
def _sc_gather(emb_hbm, idxj_hbm, out_hbm, idx_v, idxt_v, rows_v, sem):
    c = lax.axis_index("c")
    s = lax.axis_index("s")
    wid = s * NC + c
    base = wid * _EPT

    def chunk(k, _):
        b0 = base + k * _GC
        pltpu.sync_copy(idxj_hbm.at[pl.ds(b0, _GC)], idx_v)
        pltpu.async_copy(emb_hbm.at[idx_v], rows_v, sem).wait()
        pltpu.sync_copy(rows_v, out_hbm.at[pl.ds(b0, _GC)])
        return _

    lax.fori_loop(0, _GN, chunk, None)
    b0 = base + _GN * _GC
    pltpu.sync_copy(idxj_hbm.at[pl.ds(b0, _GT)], idxt_v)
    pltpu.async_copy(emb_hbm.at[idxt_v], rows_v.at[pl.ds(0, _GT)], sem).wait()
    pltpu.sync_copy(rows_v.at[pl.ds(0, _GT)], out_hbm.at[pl.ds(b0, _GT)])


_EB = 1000


def _tc_transform_body(aj_ref, f_ref, r_ref, wt_ref, b_ref, w_ref, p0_ref,
                       p1_ref, p2_ref):
    f = f_ref[...]
    w = aj_ref[...] * f
    w_ref[...] = w
    t = jnp.dot(w, wt_ref[...], preferred_element_type=jnp.float32)
    r = r_ref[...]
    rn = jnp.sqrt(jnp.sum(r * r, axis=1, keepdims=True))
    u = r / rn
    bb = b_ref[...]
    p0_ref[...] = u[:, 0:1] * t + bb
    p1_ref[...] = u[:, 1:2] * t + bb
    p2_ref[...] = u[:, 2:3] * t + bb


_tc_transform = pl.pallas_call(
    _tc_transform_body,
    grid=(E // _EB,),
    in_specs=[
        pl.BlockSpec((_EB, D), lambda i: (i, 0)),
        pl.BlockSpec((_EB, 1), lambda i: (i, 0)),
        pl.BlockSpec((_EB, 3), lambda i: (i, 0)),
        pl.BlockSpec((D, D), lambda i: (0, 0)),
        pl.BlockSpec((1, D), lambda i: (0, 0)),
    ],
    out_specs=[
        pl.BlockSpec((_EB, D), lambda i: (i, 0)),
        pl.BlockSpec((_EB, D), lambda i: (i, 0)),
        pl.BlockSpec((_EB, D), lambda i: (i, 0)),
        pl.BlockSpec((_EB, D), lambda i: (i, 0)),
    ],
    out_shape=[jax.ShapeDtypeStruct((E, D), jnp.float32)] * 4,
)


_EPS = E // NS
_SC_ = 80
_SN = _EPS // _SC_

_NR = 624
_NTAIL_BASE = 15 * _NR
_NTAIL = N - _NTAIL_BASE

_CW = 16


def _plane_copy(src, dst, s):
    nb = s * _NR

    @pl.when(s < 15)
    def _():
        pltpu.sync_copy(src.at[pl.ds(nb, _NR)], dst.at[pl.ds(nb, _NR)])

    @pl.when(s == 15)
    def _():
        pltpu.sync_copy(src.at[pl.ds(_NTAIL_BASE, _NTAIL)],
                        dst.at[pl.ds(_NTAIL_BASE, _NTAIL)])


def _accum_loop(idxi_hbm, stream_hbm, acc, idx_v, val_v, s):
    eb = s * _EPS

    def chunk(k, carry):
        b0 = eb + k * _SC_
        pltpu.sync_copy(idxi_hbm.at[pl.ds(b0, _SC_)], idx_v)
        pltpu.sync_copy(stream_hbm.at[pl.ds(b0, _SC_)], val_v)
        pltpu.sync_copy(val_v, acc.at[idx_v], add=True)
        return carry

    lax.fori_loop(0, _SN, chunk, None)


@functools.partial(
    pl.kernel,
    mesh=_sc_mesh,
    out_type=(
        jax.ShapeDtypeStruct((N, D), jnp.float32),
        jax.ShapeDtypeStruct((N, D), jnp.float32),
        jax.ShapeDtypeStruct((N, D), jnp.float32),
        jax.ShapeDtypeStruct((N, D), jnp.float32),
    ),
    scratch_types=[
        pltpu.VMEM((_SC_,), jnp.int32),
        pltpu.VMEM((_SC_, D), jnp.float32),
        pltpu.VMEM_SHARED((N, D), jnp.float32),
    ],
)
def _sc_scatter(w_hbm, p0_hbm, p1_hbm, p2_hbm, idxi_hbm, zeros_hbm,
                radial_hbm, s0_hbm, s1_hbm, s2_hbm, idx_v, val_v, acc):
    c = lax.axis_index("c")
    s = lax.axis_index("s")

    plan = [((w_hbm, radial_hbm), (p0_hbm, s0_hbm)),
            ((p1_hbm, s1_hbm), (p2_hbm, s2_hbm))]

    for (st0, out0), (st1, out1) in plan:
        _plane_copy(zeros_hbm, acc, s)
        plsc.subcore_barrier()

        @pl.when(c == 0)
        def _():
            _accum_loop(idxi_hbm, st0, acc, idx_v, val_v, s)

        @pl.when(c == 1)
        def _():
            _accum_loop(idxi_hbm, st1, acc, idx_v, val_v, s)

        plsc.subcore_barrier()

        @pl.when(c == 0)
        def _():
            _plane_copy(acc, out0, s)

        @pl.when(c == 1)
        def _():
            _plane_copy(acc, out1, s)

        plsc.subcore_barrier()


_NB = 1000


def _tc_final_body(rad_ref, s0_ref, s1_ref, s2_ref, out_ref):
    v0 = s0_ref[...]
    v1 = s1_ref[...]
    v2 = s2_ref[...]
    out_ref[:, 0:D] = jnp.sqrt(v0 * v0 + v1 * v1 + v2 * v2)
    out_ref[:, D:2 * D] = rad_ref[...]


_tc_final = pl.pallas_call(
    _tc_final_body,
    grid=(N // _NB,),
    in_specs=[
        pl.BlockSpec((_NB, D), lambda i: (i, 0)),
        pl.BlockSpec((_NB, D), lambda i: (i, 0)),
        pl.BlockSpec((_NB, D), lambda i: (i, 0)),
        pl.BlockSpec((_NB, D), lambda i: (i, 0)),
    ],
    out_specs=pl.BlockSpec((_NB, 2 * D), lambda i: (i, 0)),
    out_shape=jax.ShapeDtypeStruct((N, 2 * D), jnp.float32),
)


def kernel(atomic_embedding, pairlist, f_ij_cutoff, r_ij, W, b):
    idx_i = pairlist[0]
    idx_j = pairlist[1]
    a_j = _sc_gather(atomic_embedding, idx_j)
    w, p0, p1, p2 = _tc_transform(a_j, f_ij_cutoff, r_ij, W.T,
                                  b.reshape(1, D))
    zeros = jnp.zeros((N, D), jnp.float32)
    radial, s0, s1, s2 = _sc_scatter(w, p0, p1, p2, idx_i, zeros)
    return _tc_final(radial, s0, s1, s2)


# speedup vs baseline: 29.3717x; 1.4331x over previous
import functools

import jax
import jax.numpy as jnp
from jax import lax
from jax.experimental import pallas as pl
from jax.experimental.pallas import tpu as pltpu
from jax.experimental.pallas import tpu_sc as plsc

N = 10000
E = 160000
D = 128

NC = 2
NS = 16
NW = NC * NS


_EPT = E // NW
_GC = 128
_GN = _EPT // _GC
_GT = _EPT - _GN * _GC

_sc_mesh = plsc.VectorSubcoreMesh(core_axis_name="c", subcore_axis_name="s")


@functools.partial(
    pl.kernel,
    mesh=_sc_mesh,
    out_type=jax.ShapeDtypeStruct((E, D), jnp.float32),
    scratch_types=[
        pltpu.VMEM((_GC,), jnp.int32),
        pltpu.VMEM((_GT,), jnp.int32),
        pltpu.VMEM((_GC, D), jnp.float32),
        pltpu.SemaphoreType.DMA,
    ],
)
def _sc_gather(emb_hbm, idxj_hbm, out_hbm, idx_v, idxt_v, rows_v, sem):
    c = lax.axis_index("c")
    s = lax.axis_index("s")
    wid = s * NC + c
    base = wid * _EPT

    def chunk(k, _):
        b0 = base + k * _GC
        pltpu.sync_copy(idxj_hbm.at[pl.ds(b0, _GC)], idx_v)
        pltpu.async_copy(emb_hbm.at[idx_v], rows_v, sem).wait()
        pltpu.sync_copy(rows_v, out_hbm.at[pl.ds(b0, _GC)])
        return _

    lax.fori_loop(0, _GN, chunk, None)
    b0 = base + _GN * _GC
    pltpu.sync_copy(idxj_hbm.at[pl.ds(b0, _GT)], idxt_v)
    pltpu.async_copy(emb_hbm.at[idxt_v], rows_v.at[pl.ds(0, _GT)], sem).wait()
    pltpu.sync_copy(rows_v.at[pl.ds(0, _GT)], out_hbm.at[pl.ds(b0, _GT)])


_EB = 1000


def _tc_transform_body(aj_ref, f_ref, r_ref, wt_ref, b_ref, w_ref, p0_ref,
                       p1_ref, p2_ref):
    f = f_ref[...]
    w = aj_ref[...] * f
    w_ref[...] = w
    t = jnp.dot(w, wt_ref[...], preferred_element_type=jnp.float32)
    r = r_ref[...]
    rn = jnp.sqrt(jnp.sum(r * r, axis=1, keepdims=True))
    u = r / rn
    bb = b_ref[...]
    p0_ref[...] = u[:, 0:1] * t + bb
    p1_ref[...] = u[:, 1:2] * t + bb
    p2_ref[...] = u[:, 2:3] * t + bb


_tc_transform = pl.pallas_call(
    _tc_transform_body,
    grid=(E // _EB,),
    in_specs=[
        pl.BlockSpec((_EB, D), lambda i: (i, 0)),
        pl.BlockSpec((_EB, 1), lambda i: (i, 0)),
        pl.BlockSpec((_EB, 3), lambda i: (i, 0)),
        pl.BlockSpec((D, D), lambda i: (0, 0)),
        pl.BlockSpec((1, D), lambda i: (0, 0)),
    ],
    out_specs=[
        pl.BlockSpec((_EB, D), lambda i: (i, 0)),
        pl.BlockSpec((_EB, D), lambda i: (i, 0)),
        pl.BlockSpec((_EB, D), lambda i: (i, 0)),
        pl.BlockSpec((_EB, D), lambda i: (i, 0)),
    ],
    out_shape=[jax.ShapeDtypeStruct((E, D), jnp.float32)] * 4,
)


_EPS = E // NS
_SC_ = 80
_SN = _EPS // _SC_

_NR = 624
_NTAIL_BASE = 15 * _NR
_NTAIL = N - _NTAIL_BASE

_CW = 16


def _plane_copy(src, dst, s):
    nb = s * _NR

    @pl.when(s < 15)
    def _():
        pltpu.sync_copy(src.at[pl.ds(nb, _NR)], dst.at[pl.ds(nb, _NR)])

    @pl.when(s == 15)
    def _():
        pltpu.sync_copy(src.at[pl.ds(_NTAIL_BASE, _NTAIL)],
                        dst.at[pl.ds(_NTAIL_BASE, _NTAIL)])


def _accum_loop(idxi_hbm, stream_hbm, acc, bufs, s):
    idx0, idx1, val0, val1, si0, sv0, si1, sv1 = bufs
    eb = s * _EPS

    def fire0(k):
        b0 = eb + k * _SC_
        pltpu.async_copy(idxi_hbm.at[pl.ds(b0, _SC_)], idx0, si0)
        pltpu.async_copy(stream_hbm.at[pl.ds(b0, _SC_)], val0, sv0)

    def fire1(k):
        b0 = eb + k * _SC_
        pltpu.async_copy(idxi_hbm.at[pl.ds(b0, _SC_)], idx1, si1)
        pltpu.async_copy(stream_hbm.at[pl.ds(b0, _SC_)], val1, sv1)

    def drain0():
        pltpu.make_async_copy(idxi_hbm.at[pl.ds(eb, _SC_)], idx0, si0).wait()
        pltpu.make_async_copy(stream_hbm.at[pl.ds(eb, _SC_)], val0, sv0).wait()

    def drain1():
        pltpu.make_async_copy(idxi_hbm.at[pl.ds(eb, _SC_)], idx1, si1).wait()
        pltpu.make_async_copy(stream_hbm.at[pl.ds(eb, _SC_)], val1, sv1).wait()

    fire0(0)
    fire1(1)

    def body(g, carry):
        drain0()
        pltpu.sync_copy(val0, acc.at[idx0], add=True)
        fire0(2 * g + 2)
        drain1()
        pltpu.sync_copy(val1, acc.at[idx1], add=True)

        @pl.when(g < (_SN - 3) // 2)
        def _():
            fire1(2 * g + 3)

        return carry

    lax.fori_loop(0, (_SN - 1) // 2, body, None)
    drain0()
    pltpu.sync_copy(val0, acc.at[idx0], add=True)


@functools.partial(
    pl.kernel,
    mesh=_sc_mesh,
    out_type=(
        jax.ShapeDtypeStruct((N, D), jnp.float32),
        jax.ShapeDtypeStruct((N, D), jnp.float32),
        jax.ShapeDtypeStruct((N, D), jnp.float32),
        jax.ShapeDtypeStruct((N, D), jnp.float32),
    ),
    scratch_types=[
        pltpu.VMEM((_SC_,), jnp.int32),
        pltpu.VMEM((_SC_,), jnp.int32),
        pltpu.VMEM((_SC_, D), jnp.float32),
        pltpu.VMEM((_SC_, D), jnp.float32),
        pltpu.SemaphoreType.DMA,
        pltpu.SemaphoreType.DMA,
        pltpu.SemaphoreType.DMA,
        pltpu.SemaphoreType.DMA,
        pltpu.VMEM_SHARED((N, D), jnp.float32),
    ],
)
def _sc_scatter(w_hbm, p0_hbm, p1_hbm, p2_hbm, idxi_hbm, zeros_hbm,
                radial_hbm, s0_hbm, s1_hbm, s2_hbm, idx0, idx1, val0, val1,
                si0, sv0, si1, sv1, acc):
    c = lax.axis_index("c")
    s = lax.axis_index("s")
    bufs = (idx0, idx1, val0, val1, si0, sv0, si1, sv1)

    plan = [((w_hbm, radial_hbm), (p0_hbm, s0_hbm)),
            ((p1_hbm, s1_hbm), (p2_hbm, s2_hbm))]

    for (st0, out0), (st1, out1) in plan:
        _plane_copy(zeros_hbm, acc, s)
        plsc.subcore_barrier()

        @pl.when(c == 0)
        def _():
            _accum_loop(idxi_hbm, st0, acc, bufs, s)

        @pl.when(c == 1)
        def _():
            _accum_loop(idxi_hbm, st1, acc, bufs, s)

        plsc.subcore_barrier()

        @pl.when(c == 0)
        def _():
            _plane_copy(acc, out0, s)

        @pl.when(c == 1)
        def _():
            _plane_copy(acc, out1, s)

        plsc.subcore_barrier()


_NB = 1000


def _tc_final_body(rad_ref, s0_ref, s1_ref, s2_ref, out_ref):
    v0 = s0_ref[...]
    v1 = s1_ref[...]
    v2 = s2_ref[...]
    out_ref[:, 0:D] = jnp.sqrt(v0 * v0 + v1 * v1 + v2 * v2)
    out_ref[:, D:2 * D] = rad_ref[...]


_tc_final = pl.pallas_call(
    _tc_final_body,
    grid=(N // _NB,),
    in_specs=[
        pl.BlockSpec((_NB, D), lambda i: (i, 0)),
        pl.BlockSpec((_NB, D), lambda i: (i, 0)),
        pl.BlockSpec((_NB, D), lambda i: (i, 0)),
        pl.BlockSpec((_NB, D), lambda i: (i, 0)),
    ],
    out_specs=pl.BlockSpec((_NB, 2 * D), lambda i: (i, 0)),
    out_shape=jax.ShapeDtypeStruct((N, 2 * D), jnp.float32),
)


def kernel(atomic_embedding, pairlist, f_ij_cutoff, r_ij, W, b):
    idx_i = pairlist[0]
    idx_j = pairlist[1]
    a_j = _sc_gather(atomic_embedding, idx_j)
    w, p0, p1, p2 = _tc_transform(a_j, f_ij_cutoff, r_ij, W.T,
                                  b.reshape(1, D))
    zeros = jnp.zeros((N, D), jnp.float32)
    radial, s0, s1, s2 = _sc_scatter(w, p0, p1, p2, idx_i, zeros)
    return _tc_final(radial, s0, s1, s2)
